# baseline (speedup 1.0000x reference)
import functools

import jax
import jax.numpy as jnp
from jax import lax
from jax.experimental import pallas as pl
from jax.experimental.pallas import tpu as pltpu
from jax.experimental.pallas import tpu_sc as plsc

_N = 10000
_E = 160000
_D = 256
_H = 128
_K = 80
_C = 125
_T = 16
_NA = 10240
_WB = 10160
_RPA = _NA // _T
_BK = 1000


def _sc_agg(h_tbl, pk3, ppr3):
    mesh = plsc.VectorSubcoreMesh(core_axis_name="c", subcore_axis_name="s")

    @functools.partial(
        pl.kernel,
        mesh=mesh,
        out_type=jax.ShapeDtypeStruct((2, _NA, _H), jnp.float32),
        scratch_types=[
            pltpu.VMEM((3, _K), jnp.int32),
            pltpu.VMEM((3, _K), jnp.float32),
            pltpu.VMEM((3, _K), jnp.int32),
            pltpu.VMEM((3, _K), jnp.int32),
            pltpu.VMEM((3, _K, _H), jnp.float32),
            pltpu.VMEM((_K, _H), jnp.float32),
            pltpu.VMEM((1, _K), jnp.int32),
            pltpu.VMEM_SHARED((_NA, _H), jnp.float32),
            pltpu.SemaphoreType.DMA,
            pltpu.SemaphoreType.DMA,
            pltpu.SemaphoreType.DMA,
            pltpu.SemaphoreType.DMA,
            pltpu.SemaphoreType.DMA,
            pltpu.SemaphoreType.DMA,
            pltpu.SemaphoreType.DMA,
            pltpu.SemaphoreType.DMA,
            pltpu.SemaphoreType.DMA,
            pltpu.SemaphoreType.DMA,
            pltpu.SemaphoreType.DMA,
            pltpu.SemaphoreType.DMA,
        ],
    )
    def k(h_hbm, pk_hbm, ppr_hbm, out_hbm,
          pk_v, ppr_v, idx_v, dst_v, rows_v, wp_v, widx_v, acc_sh,
          ek0, ek1, ek2, ep0, ep1, ep2, gs0, gs1, gs2, ss0, ss1, ss2):
        c = lax.axis_index("c")
        s = lax.axis_index("s")
        cofs = c * _N
        eks = (ek0, ek1, ek2)
        eps = (ep0, ep1, ep2)
        gss = (gs0, gs1, gs2)
        sss = (ss0, ss1, ss2)
        lane = lax.iota(jnp.int32, 16)

        def start_edge(t, e):
            ofs = (s * _C + t) * _K
            pltpu.async_copy(pk_hbm.at[pl.ds(ofs, _K)], pk_v.at[e], eks[e])
            pltpu.async_copy(ppr_hbm.at[pl.ds(ofs, _K)], ppr_v.at[e], eps[e])

        def wait_edge(e):
            pltpu.make_async_copy(
                pk_hbm.at[pl.ds(0, _K)], pk_v.at[e], eks[e]).wait()
            pltpu.make_async_copy(
                ppr_hbm.at[pl.ds(0, _K)], ppr_v.at[e], eps[e]).wait()

        def unpack(e, b):
            for u in range(_K // 16):
                sl = pl.ds(u * 16, 16)
                pk = pk_v[e, sl]
                idx_v[b, sl] = (pk & 0xFFFF) + cofs
                dst_v[b, sl] = pk >> 16

        def start_gather(b):
            pltpu.async_copy(h_hbm.at[idx_v.at[b]], rows_v.at[b], gss[b])

        def wait_gather(b):
            pltpu.make_async_copy(
                h_hbm.at[idx_v.at[b]], rows_v.at[b], gss[b]).wait()

        def start_scatter(b):
            pltpu.async_copy(rows_v.at[b], acc_sh.at[dst_v.at[b]], sss[b],
                             add=True)

        def wait_scatter(b):
            pltpu.make_async_copy(
                rows_v.at[b], acc_sh.at[dst_v.at[b]], sss[b]).wait()

        def scale(e, b):
            def sgroup(g, gcarry):
                sl16 = pl.ds(g * 16, 16)
                pw = ppr_v[e, sl16]
                dv = dst_v[b, sl16]
                rbase = g * 16
                for rr in range(16):
                    pvs = pw[rr]
                    pv = jnp.full((16,), pvs, jnp.float32)
                    for u in range(_H // 16):
                        sl = pl.ds(u * 16, 16)
                        rows_v[b, rbase + rr, sl] = (
                            rows_v[b, rbase + rr, sl] * pv)
                    dvs = dv[rr]
                    wr = dvs >> 7
                    wc = dvs & 0x70
                    wl = dvs & 15
                    wsl = pl.ds(wc, 16)
                    v = wp_v[wr, wsl]
                    wp_v[wr, wsl] = jnp.where(lane == wl, v + pvs, v)
                return gcarry

            lax.fori_loop(0, _K // 16, sgroup, 0)

        zero16f = jnp.zeros((16,), jnp.float32)
        zero16i = jnp.zeros((16,), jnp.int32)

        def zrow(r, carry):
            for u in range(_H // 16):
                sl = pl.ds(u * 16, 16)
                rows_v[0, r, sl] = zero16f
                rows_v[2, r, sl] = zero16f
                wp_v[r, sl] = zero16f
            return carry

        lax.fori_loop(0, _K, zrow, 0)
        for u in range(_K // 16):
            sl = pl.ds(u * 16, 16)
            dst_v[2, sl] = zero16i
            widx_v[0, sl] = lane + (_WB + u * 16)

        base = s * _RPA
        for blk in range(_RPA // _K):
            pltpu.sync_copy(rows_v.at[0], acc_sh.at[pl.ds(base + blk * _K, _K)])

        for t0 in range(3):
            start_edge(t0, t0)

        plsc.subcore_barrier()

        start_scatter(2)

        wait_edge(0)
        unpack(0, 0)
        start_gather(0)
        wait_edge(1)
        unpack(1, 1)
        start_gather(1)

        def step(t, m3, prep, estart):
            b = m3
            b2 = (m3 + 2) % 3
            if prep:
                wait_edge(b2)
                wait_scatter(b2)
                unpack(b2, b2)
                start_gather(b2)
            wait_gather(b)
            scale(b, b)
            if estart:

                @pl.when(t + 3 < _C)
                def _():
                    start_edge(t + 3, b)

            start_scatter(b)

        def triple(i, carry):
            t = i * 3
            step(t, 0, True, True)
            step(t + 1, 1, True, True)
            step(t + 2, 2, True, True)
            return carry

        lax.fori_loop(0, 40, triple, 0)

        step(120, 0, True, True)
        step(121, 1, True, True)
        step(122, 2, True, True)
        step(123, 0, False, False)
        step(124, 1, False, False)

        wait_scatter(2)
        wait_scatter(0)
        wait_scatter(1)

        pltpu.sync_copy(wp_v, acc_sh.at[widx_v.at[0]], add=True)

        plsc.subcore_barrier()

        for blk in range(_RPA // _K):
            ofs = base + blk * _K
            pltpu.sync_copy(acc_sh.at[pl.ds(ofs, _K)],
                            out_hbm.at[c, pl.ds(ofs, _K)])

    return k(h_tbl, pk3, ppr3)


def _tc_body(hh_ref, agg_ref, w_ref, ws_ref, wl_ref, wh_ref, b_ref, o_ref,
             aux_ref=None):
    dn = (((1,), (1,)), ((), ()))
    w = w_ref[...]
    inv = 1.0 / jnp.where(w == 0.0, 1.0, w)
    alo = agg_ref[0] * inv
    ahi = agg_ref[1] * inv
    acc = (lax.dot_general(hh_ref[...], ws_ref[...], dn,
                           preferred_element_type=jnp.float32)
           + lax.dot_general(alo, wl_ref[...], dn,
                             preferred_element_type=jnp.float32)
           + lax.dot_general(ahi, wh_ref[...], dn,
                             preferred_element_type=jnp.float32)
           + b_ref[...])
    y = jnp.where(acc > 0.0, acc, 0.01 * acc)
    ss = jnp.sum(y * y, axis=1, keepdims=True)
    nrm = jnp.sqrt(ss)
    nrm = jnp.where(nrm == 0.0, 1.0, nrm)
    res = y / nrm
    o_ref[...] = res
    if aux_ref is not None:
        aux_ref[0] = res[:, 0:128]
        aux_ref[1] = res[:, 128:256]


def _tc_update(hh, agg, w, W, b, make_aux):
    grid = (_N // _BK,)
    in_specs = [
        pl.BlockSpec((_BK, _D), lambda i: (i, 0)),
        pl.BlockSpec((2, _BK, _H), lambda i: (0, i, 0)),
        pl.BlockSpec((_BK, 1), lambda i: (i, 0)),
        pl.BlockSpec((_D, _D), lambda i: (0, 0)),
        pl.BlockSpec((_D, 128), lambda i: (0, 2)),
        pl.BlockSpec((_D, 128), lambda i: (0, 3)),
        pl.BlockSpec((1, _D), lambda i: (0, 0)),
    ]
    out_shape = [jax.ShapeDtypeStruct((_N, _D), jnp.float32)]
    out_specs = [pl.BlockSpec((_BK, _D), lambda i: (i, 0))]
    if make_aux:
        out_shape.append(jax.ShapeDtypeStruct((2, _N, _H), jnp.float32))
        out_specs.append(pl.BlockSpec((2, _BK, _H), lambda i: (0, i, 0)))
    return pl.pallas_call(
        _tc_body,
        grid=grid,
        in_specs=in_specs,
        out_specs=out_specs,
        out_shape=out_shape,
    )(hh, agg, w, W, W, W, b.reshape(1, _D))


def _prep_body(h_ref, o_ref):
    o_ref[0] = h_ref[:, 0:128]
    o_ref[1] = h_ref[:, 128:256]


def _prep(h):
    return pl.pallas_call(
        _prep_body,
        grid=(_N // _BK,),
        in_specs=[pl.BlockSpec((_BK, _D), lambda i: (i, 0))],
        out_specs=pl.BlockSpec((2, _BK, _H), lambda i: (0, i, 0)),
        out_shape=jax.ShapeDtypeStruct((2, _N, _H), jnp.float32),
    )(h)


def _wsum(agg):
    return agg[0, _WB:_NA].reshape(-1)[:_N, None]


def kernel(h, edge_index, ppr_weight, W1, b1, W2, b2):
    pk3 = edge_index[0] | (edge_index[1] << 16)
    ppr3 = ppr_weight

    tbl = _prep(h)
    agg1 = _sc_agg(tbl.reshape(2 * _N, _H), pk3, ppr3)
    hh1, tbl1 = _tc_update(h, agg1, _wsum(agg1), W1, b1, make_aux=True)
    agg2 = _sc_agg(tbl1.reshape(2 * _N, _H), pk3, ppr3)
    (hh2,) = _tc_update(hh1, agg2, _wsum(agg2), W2, b2, make_aux=False)
    return hh2


# speedup vs baseline: 1.4030x; 1.4030x over previous
import functools

import jax
import jax.numpy as jnp
from jax import lax
from jax.experimental import pallas as pl
from jax.experimental.pallas import tpu as pltpu
from jax.experimental.pallas import tpu_sc as plsc

_N = 10000
_E = 160000
_D = 256
_H = 128
_K = 80
_C = 125
_T = 16
_NA = 10240
_RPA = _NA // _T
_BK = 1000


def _sc_agg(h_tbl, pk3, ppr3):
    mesh = plsc.VectorSubcoreMesh(core_axis_name="c", subcore_axis_name="s")

    @functools.partial(
        pl.kernel,
        mesh=mesh,
        out_type=jax.ShapeDtypeStruct((2, _NA, _H), jnp.float32),
        scratch_types=[
            pltpu.VMEM((3, _K), jnp.int32),
            pltpu.VMEM((3, _K), jnp.float32),
            pltpu.VMEM((3, _K), jnp.int32),
            pltpu.VMEM((3, _K), jnp.int32),
            pltpu.VMEM((3, _K, _H), jnp.float32),
            pltpu.VMEM_SHARED((_NA, _H), jnp.float32),
            pltpu.SemaphoreType.DMA,
            pltpu.SemaphoreType.DMA,
            pltpu.SemaphoreType.DMA,
            pltpu.SemaphoreType.DMA,
            pltpu.SemaphoreType.DMA,
            pltpu.SemaphoreType.DMA,
            pltpu.SemaphoreType.DMA,
            pltpu.SemaphoreType.DMA,
            pltpu.SemaphoreType.DMA,
            pltpu.SemaphoreType.DMA,
            pltpu.SemaphoreType.DMA,
            pltpu.SemaphoreType.DMA,
        ],
    )
    def k(h_hbm, pk_hbm, ppr_hbm, out_hbm,
          pk_v, ppr_v, idx_v, dst_v, rows_v, acc_sh,
          ek0, ek1, ek2, ep0, ep1, ep2, gs0, gs1, gs2, ss0, ss1, ss2):
        c = lax.axis_index("c")
        s = lax.axis_index("s")
        cofs = c * _N
        eks = (ek0, ek1, ek2)
        eps = (ep0, ep1, ep2)
        gss = (gs0, gs1, gs2)
        sss = (ss0, ss1, ss2)

        def start_edge(t, e):
            ofs = (s * _C + t) * _K
            pltpu.async_copy(pk_hbm.at[pl.ds(ofs, _K)], pk_v.at[e], eks[e])
            pltpu.async_copy(ppr_hbm.at[pl.ds(ofs, _K)], ppr_v.at[e], eps[e])

        def wait_edge(e):
            pltpu.make_async_copy(
                pk_hbm.at[pl.ds(0, _K)], pk_v.at[e], eks[e]).wait()
            pltpu.make_async_copy(
                ppr_hbm.at[pl.ds(0, _K)], ppr_v.at[e], eps[e]).wait()

        def unpack(e, b):
            for u in range(_K // 16):
                sl = pl.ds(u * 16, 16)
                pk = pk_v[e, sl]
                idx_v[b, sl] = (pk & 0xFFFF) + cofs
                dst_v[b, sl] = pk >> 16

        def start_gather(b):
            pltpu.async_copy(h_hbm.at[idx_v.at[b]], rows_v.at[b], gss[b])

        def wait_gather(b):
            pltpu.make_async_copy(
                h_hbm.at[idx_v.at[b]], rows_v.at[b], gss[b]).wait()

        def start_scatter(b):
            pltpu.async_copy(rows_v.at[b], acc_sh.at[dst_v.at[b]], sss[b],
                             add=True)

        def wait_scatter(b):
            pltpu.make_async_copy(
                rows_v.at[b], acc_sh.at[dst_v.at[b]], sss[b]).wait()

        def scale(e, b):
            def sgroup(g, gcarry):
                pw = ppr_v[e, pl.ds(g * 16, 16)]
                rbase = g * 16
                for rr in range(16):
                    pv = jnp.full((16,), pw[rr], jnp.float32)
                    for u in range(_H // 16):
                        sl = pl.ds(u * 16, 16)
                        rows_v[b, rbase + rr, sl] = (
                            rows_v[b, rbase + rr, sl] * pv)
                return gcarry

            lax.fori_loop(0, _K // 16, sgroup, 0)

        zero16f = jnp.zeros((16,), jnp.float32)
        zero16i = jnp.zeros((16,), jnp.int32)

        def zrow(r, carry):
            for u in range(_H // 16):
                sl = pl.ds(u * 16, 16)
                rows_v[0, r, sl] = zero16f
                rows_v[2, r, sl] = zero16f
            return carry

        lax.fori_loop(0, _K, zrow, 0)
        for u in range(_K // 16):
            dst_v[2, pl.ds(u * 16, 16)] = zero16i

        base = s * _RPA
        for blk in range(_RPA // _K):
            pltpu.sync_copy(rows_v.at[0], acc_sh.at[pl.ds(base + blk * _K, _K)])

        for t0 in range(3):
            start_edge(t0, t0)

        plsc.subcore_barrier()

        start_scatter(2)

        wait_edge(0)
        unpack(0, 0)
        start_gather(0)
        wait_edge(1)
        unpack(1, 1)
        start_gather(1)

        def step(t, m3, prep, estart):
            b = m3
            b2 = (m3 + 2) % 3
            if prep:
                wait_edge(b2)
                wait_scatter(b2)
                unpack(b2, b2)
                start_gather(b2)
            wait_gather(b)
            scale(b, b)
            if estart:

                @pl.when(t + 3 < _C)
                def _():
                    start_edge(t + 3, b)

            start_scatter(b)

        def triple(i, carry):
            t = i * 3
            step(t, 0, True, True)
            step(t + 1, 1, True, True)
            step(t + 2, 2, True, True)
            return carry

        lax.fori_loop(0, 40, triple, 0)

        step(120, 0, True, True)
        step(121, 1, True, True)
        step(122, 2, True, True)
        step(123, 0, False, False)
        step(124, 1, False, False)

        wait_scatter(2)
        wait_scatter(0)
        wait_scatter(1)

        plsc.subcore_barrier()

        for blk in range(_RPA // _K):
            ofs = base + blk * _K
            pltpu.sync_copy(acc_sh.at[pl.ds(ofs, _K)],
                            out_hbm.at[c, pl.ds(ofs, _K)])

    return k(h_tbl, pk3, ppr3)


_EPW = _E // 32


def _sc_wsum(pk3, ppr3):
    mesh = plsc.VectorSubcoreMesh(core_axis_name="c", subcore_axis_name="s")

    @functools.partial(
        pl.kernel,
        mesh=mesh,
        compiler_params=pltpu.CompilerParams(use_tc_tiling_on_sc=False),
        out_type=jax.ShapeDtypeStruct((2, _NA // _H, _H), jnp.float32),
        scratch_types=[
            pltpu.VMEM((_EPW + 16,), jnp.int32),
            pltpu.VMEM((_EPW + 16,), jnp.float32),
            pltpu.VMEM((_NA // _H, _H), jnp.float32),
            pltpu.VMEM((1, _NA // _H), jnp.int32),
            pltpu.VMEM_SHARED((_NA // _H, _H), jnp.float32),
        ],
    )
    def k(pk_hbm, ppr_hbm, out_hbm, pk_v, ppr_v, wp_v, widx_v, acc_sh):
        c = lax.axis_index("c")
        s = lax.axis_index("s")
        wid = c * _T + s
        lane = lax.iota(jnp.int32, 16)
        nrows = _NA // _H

        zero16f = jnp.zeros((16,), jnp.float32)

        def zrow(r, carry):
            for u in range(_H // 16):
                wp_v[r, pl.ds(u * 16, 16)] = zero16f
            return carry

        lax.fori_loop(0, nrows, zrow, 0)
        for u in range(nrows // 16):
            widx_v[0, pl.ds(u * 16, 16)] = lane + u * 16

        pltpu.sync_copy(wp_v.at[pl.ds(0, nrows // _T)],
                        acc_sh.at[pl.ds(s * (nrows // _T), nrows // _T)])

        eofs = wid * _EPW
        pltpu.sync_copy(pk_hbm.at[pl.ds(eofs, _EPW)],
                        pk_v.at[pl.ds(0, _EPW)])
        pltpu.sync_copy(ppr_hbm.at[pl.ds(eofs, _EPW)],
                        ppr_v.at[pl.ds(0, _EPW)])

        plsc.subcore_barrier()

        ngrp = _EPW // 16
        tail = _EPW - ngrp * 16

        def edge_add(dv, pv, rr):
            dvs = dv[rr]
            pvs = pv[rr]
            wr = dvs >> 7
            wc = dvs & 0x70
            wl = dvs & 15
            wsl = pl.ds(wc, 16)
            v = wp_v[wr, wsl]
            wp_v[wr, wsl] = jnp.where(lane == wl, v + pvs, v)

        def grp(i, carry):
            sl = pl.ds(i * 16, 16)
            dv = pk_v[sl] >> 16
            pv = ppr_v[sl]
            for rr in range(16):
                edge_add(dv, pv, rr)
            return carry

        lax.fori_loop(0, ngrp, grp, 0)

        sl = pl.ds(ngrp * 16, 16)
        dv = pk_v[sl] >> 16
        pv = ppr_v[sl]
        for rr in range(tail):
            edge_add(dv, pv, rr)

        pltpu.sync_copy(wp_v, acc_sh.at[widx_v.at[0]], add=True)

        plsc.subcore_barrier()

        @pl.when(s == 0)
        def _():
            pltpu.sync_copy(acc_sh, out_hbm.at[c])

    return k(pk3, ppr3)


def _tc_body(hh_ref, agg_ref, w_ref, ws_ref, wl_ref, wh_ref, b_ref, o_ref,
             aux_ref=None):
    dn = (((1,), (1,)), ((), ()))
    w = w_ref[...]
    inv = 1.0 / jnp.where(w == 0.0, 1.0, w)
    alo = agg_ref[0] * inv
    ahi = agg_ref[1] * inv
    acc = (lax.dot_general(hh_ref[...], ws_ref[...], dn,
                           preferred_element_type=jnp.float32)
           + lax.dot_general(alo, wl_ref[...], dn,
                             preferred_element_type=jnp.float32)
           + lax.dot_general(ahi, wh_ref[...], dn,
                             preferred_element_type=jnp.float32)
           + b_ref[...])
    y = jnp.where(acc > 0.0, acc, 0.01 * acc)
    ss = jnp.sum(y * y, axis=1, keepdims=True)
    nrm = jnp.sqrt(ss)
    nrm = jnp.where(nrm == 0.0, 1.0, nrm)
    res = y / nrm
    o_ref[...] = res
    if aux_ref is not None:
        aux_ref[0] = res[:, 0:128]
        aux_ref[1] = res[:, 128:256]


def _tc_update(hh, agg, w, W, b, make_aux):
    grid = (_N // _BK,)
    in_specs = [
        pl.BlockSpec((_BK, _D), lambda i: (i, 0)),
        pl.BlockSpec((2, _BK, _H), lambda i: (0, i, 0)),
        pl.BlockSpec((_BK, 1), lambda i: (i, 0)),
        pl.BlockSpec((_D, _D), lambda i: (0, 0)),
        pl.BlockSpec((_D, 128), lambda i: (0, 2)),
        pl.BlockSpec((_D, 128), lambda i: (0, 3)),
        pl.BlockSpec((1, _D), lambda i: (0, 0)),
    ]
    out_shape = [jax.ShapeDtypeStruct((_N, _D), jnp.float32)]
    out_specs = [pl.BlockSpec((_BK, _D), lambda i: (i, 0))]
    if make_aux:
        out_shape.append(jax.ShapeDtypeStruct((2, _N, _H), jnp.float32))
        out_specs.append(pl.BlockSpec((2, _BK, _H), lambda i: (0, i, 0)))
    return pl.pallas_call(
        _tc_body,
        grid=grid,
        in_specs=in_specs,
        out_specs=out_specs,
        out_shape=out_shape,
    )(hh, agg, w, W, W, W, b.reshape(1, _D))


def _prep_body(h_ref, o_ref):
    o_ref[0] = h_ref[:, 0:128]
    o_ref[1] = h_ref[:, 128:256]


def _prep(h):
    return pl.pallas_call(
        _prep_body,
        grid=(_N // _BK,),
        in_specs=[pl.BlockSpec((_BK, _D), lambda i: (i, 0))],
        out_specs=pl.BlockSpec((2, _BK, _H), lambda i: (0, i, 0)),
        out_shape=jax.ShapeDtypeStruct((2, _N, _H), jnp.float32),
    )(h)


def kernel(h, edge_index, ppr_weight, W1, b1, W2, b2):
    pk3 = edge_index[0] | (edge_index[1] << 16)
    ppr3 = ppr_weight

    wpair = _sc_wsum(pk3, ppr3)
    w = (wpair[0] + wpair[1]).reshape(-1)[:_N, None]
    tbl = _prep(h)
    agg1 = _sc_agg(tbl.reshape(2 * _N, _H), pk3, ppr3)
    hh1, tbl1 = _tc_update(h, agg1, w, W1, b1, make_aux=True)
    agg2 = _sc_agg(tbl1.reshape(2 * _N, _H), pk3, ppr3)
    (hh2,) = _tc_update(hh1, agg2, w, W2, b2, make_aux=False)
    return hh2


# speedup vs baseline: 1.5613x; 1.1128x over previous
import functools

import jax
import jax.numpy as jnp
from jax import lax
from jax.experimental import pallas as pl
from jax.experimental.pallas import tpu as pltpu
from jax.experimental.pallas import tpu_sc as plsc

_N = 10000
_E = 160000
_D = 256
_H = 128
_K = 80
_C = 125
_T = 16
_NA = 10240
_RPA = _NA // _T
_BK = 1000


def _sc_agg(h_tbl, pk3, ppr3):
    mesh = plsc.VectorSubcoreMesh(core_axis_name="c", subcore_axis_name="s")

    @functools.partial(
        pl.kernel,
        mesh=mesh,
        out_type=jax.ShapeDtypeStruct((2, _NA, _H), jnp.float32),
        scratch_types=[
            pltpu.VMEM((4, _K), jnp.int32),
            pltpu.VMEM((4, _K), jnp.float32),
            pltpu.VMEM((4, _K), jnp.int32),
            pltpu.VMEM((4, _K), jnp.int32),
            pltpu.VMEM((4, _K, _H), jnp.float32),
            pltpu.VMEM_SHARED((_NA, _H), jnp.float32),
            pltpu.SemaphoreType.DMA,
            pltpu.SemaphoreType.DMA,
            pltpu.SemaphoreType.DMA,
            pltpu.SemaphoreType.DMA,
            pltpu.SemaphoreType.DMA,
            pltpu.SemaphoreType.DMA,
            pltpu.SemaphoreType.DMA,
            pltpu.SemaphoreType.DMA,
            pltpu.SemaphoreType.DMA,
            pltpu.SemaphoreType.DMA,
            pltpu.SemaphoreType.DMA,
            pltpu.SemaphoreType.DMA,
            pltpu.SemaphoreType.DMA,
            pltpu.SemaphoreType.DMA,
            pltpu.SemaphoreType.DMA,
            pltpu.SemaphoreType.DMA,
        ],
    )
    def k(h_hbm, pk_hbm, ppr_hbm, out_hbm,
          pk_v, ppr_v, idx_v, dst_v, rows_v, acc_sh,
          ek0, ek1, ek2, ek3, ep0, ep1, ep2, ep3,
          gs0, gs1, gs2, gs3, ss0, ss1, ss2, ss3):
        c = lax.axis_index("c")
        s = lax.axis_index("s")
        cofs = c * _N
        eks = (ek0, ek1, ek2, ek3)
        eps = (ep0, ep1, ep2, ep3)
        gss = (gs0, gs1, gs2, gs3)
        sss = (ss0, ss1, ss2, ss3)

        def start_edge(t, e):
            ofs = (s * _C + t) * _K
            pltpu.async_copy(pk_hbm.at[pl.ds(ofs, _K)], pk_v.at[e], eks[e])
            pltpu.async_copy(ppr_hbm.at[pl.ds(ofs, _K)], ppr_v.at[e], eps[e])

        def wait_edge(e):
            pltpu.make_async_copy(
                pk_hbm.at[pl.ds(0, _K)], pk_v.at[e], eks[e]).wait()
            pltpu.make_async_copy(
                ppr_hbm.at[pl.ds(0, _K)], ppr_v.at[e], eps[e]).wait()

        def unpack(e, b):
            for u in range(_K // 16):
                sl = pl.ds(u * 16, 16)
                pk = pk_v[e, sl]
                idx_v[b, sl] = (pk & 0xFFFF) + cofs
                dst_v[b, sl] = pk >> 16

        def start_gather(b):
            pltpu.async_copy(h_hbm.at[idx_v.at[b]], rows_v.at[b], gss[b])

        def wait_gather(b):
            pltpu.make_async_copy(
                h_hbm.at[idx_v.at[b]], rows_v.at[b], gss[b]).wait()

        def start_scatter(b):
            pltpu.async_copy(rows_v.at[b], acc_sh.at[dst_v.at[b]], sss[b],
                             add=True)

        def wait_scatter(b):
            pltpu.make_async_copy(
                rows_v.at[b], acc_sh.at[dst_v.at[b]], sss[b]).wait()

        def scale(e, b):
            def sgroup(g, gcarry):
                pw = ppr_v[e, pl.ds(g * 16, 16)]
                rbase = g * 16
                for rr in range(16):
                    pv = jnp.full((16,), pw[rr], jnp.float32)
                    for u in range(_H // 16):
                        sl = pl.ds(u * 16, 16)
                        rows_v[b, rbase + rr, sl] = (
                            rows_v[b, rbase + rr, sl] * pv)
                return gcarry

            lax.fori_loop(0, _K // 16, sgroup, 0)

        zero16f = jnp.zeros((16,), jnp.float32)
        zero16i = jnp.zeros((16,), jnp.int32)

        def zrow(r, carry):
            for u in range(_H // 16):
                sl = pl.ds(u * 16, 16)
                rows_v[0, r, sl] = zero16f
                rows_v[2, r, sl] = zero16f
                rows_v[3, r, sl] = zero16f
            return carry

        lax.fori_loop(0, _K, zrow, 0)
        for u in range(_K // 16):
            dst_v[2, pl.ds(u * 16, 16)] = zero16i
            dst_v[3, pl.ds(u * 16, 16)] = zero16i

        base = s * _RPA
        for blk in range(_RPA // _K):
            pltpu.sync_copy(rows_v.at[0], acc_sh.at[pl.ds(base + blk * _K, _K)])

        for t0 in range(4):
            start_edge(t0, t0)

        plsc.subcore_barrier()

        start_scatter(2)
        start_scatter(3)

        wait_edge(0)
        unpack(0, 0)
        start_gather(0)
        wait_edge(1)
        unpack(1, 1)
        start_gather(1)

        def step(t, m4, prep, estart):
            b = m4
            b2 = (m4 + 2) % 4
            if prep:
                wait_edge(b2)
                wait_scatter(b2)
                unpack(b2, b2)
                start_gather(b2)
            wait_gather(b)
            scale(b, b)
            if estart:

                @pl.when(t + 4 < _C)
                def _():
                    start_edge(t + 4, b)

            start_scatter(b)

        def quad(i, carry):
            t = i * 4
            step(t, 0, True, True)
            step(t + 1, 1, True, True)
            step(t + 2, 2, True, True)
            step(t + 3, 3, True, True)
            return carry

        lax.fori_loop(0, 30, quad, 0)

        step(120, 0, True, True)
        step(121, 1, True, True)
        step(122, 2, True, True)
        step(123, 3, False, False)
        step(124, 0, False, False)

        wait_scatter(1)
        wait_scatter(2)
        wait_scatter(3)
        wait_scatter(0)

        plsc.subcore_barrier()

        for blk in range(_RPA // _K):
            ofs = base + blk * _K
            pltpu.sync_copy(acc_sh.at[pl.ds(ofs, _K)],
                            out_hbm.at[c, pl.ds(ofs, _K)])

    return k(h_tbl, pk3, ppr3)


_EPW = _E // 32


def _sc_wsum(pk3, ppr3):
    mesh = plsc.VectorSubcoreMesh(core_axis_name="c", subcore_axis_name="s")

    @functools.partial(
        pl.kernel,
        mesh=mesh,
        compiler_params=pltpu.CompilerParams(use_tc_tiling_on_sc=False),
        out_type=jax.ShapeDtypeStruct((2, _NA // _H, _H), jnp.float32),
        scratch_types=[
            pltpu.VMEM((_EPW + 16,), jnp.int32),
            pltpu.VMEM((_EPW + 16,), jnp.float32),
            pltpu.VMEM((_NA // _H, _H), jnp.float32),
            pltpu.VMEM((1, _NA // _H), jnp.int32),
            pltpu.VMEM_SHARED((_NA // _H, _H), jnp.float32),
        ],
    )
    def k(pk_hbm, ppr_hbm, out_hbm, pk_v, ppr_v, wp_v, widx_v, acc_sh):
        c = lax.axis_index("c")
        s = lax.axis_index("s")
        wid = c * _T + s
        lane = lax.iota(jnp.int32, 16)
        nrows = _NA // _H

        zero16f = jnp.zeros((16,), jnp.float32)

        def zrow(r, carry):
            for u in range(_H // 16):
                wp_v[r, pl.ds(u * 16, 16)] = zero16f
            return carry

        lax.fori_loop(0, nrows, zrow, 0)
        for u in range(nrows // 16):
            widx_v[0, pl.ds(u * 16, 16)] = lane + u * 16

        pltpu.sync_copy(wp_v.at[pl.ds(0, nrows // _T)],
                        acc_sh.at[pl.ds(s * (nrows // _T), nrows // _T)])

        eofs = wid * _EPW
        pltpu.sync_copy(pk_hbm.at[pl.ds(eofs, _EPW)],
                        pk_v.at[pl.ds(0, _EPW)])
        pltpu.sync_copy(ppr_hbm.at[pl.ds(eofs, _EPW)],
                        ppr_v.at[pl.ds(0, _EPW)])

        plsc.subcore_barrier()

        ngrp = _EPW // 16
        tail = _EPW - ngrp * 16

        def edge_add(dv, pv, rr):
            dvs = dv[rr]
            pvs = pv[rr]
            wr = dvs >> 7
            wc = dvs & 0x70
            wl = dvs & 15
            wsl = pl.ds(wc, 16)
            v = wp_v[wr, wsl]
            wp_v[wr, wsl] = jnp.where(lane == wl, v + pvs, v)

        def grp(i, carry):
            sl = pl.ds(i * 16, 16)
            dv = pk_v[sl] >> 16
            pv = ppr_v[sl]
            for rr in range(16):
                edge_add(dv, pv, rr)
            return carry

        lax.fori_loop(0, ngrp, grp, 0)

        sl = pl.ds(ngrp * 16, 16)
        dv = pk_v[sl] >> 16
        pv = ppr_v[sl]
        for rr in range(tail):
            edge_add(dv, pv, rr)

        pltpu.sync_copy(wp_v, acc_sh.at[widx_v.at[0]], add=True)

        plsc.subcore_barrier()

        @pl.when(s == 0)
        def _():
            pltpu.sync_copy(acc_sh, out_hbm.at[c])

    return k(pk3, ppr3)


def _tc_body(hh_ref, agg_ref, w_ref, ws_ref, wl_ref, wh_ref, b_ref, o_ref,
             aux_ref=None):
    dn = (((1,), (1,)), ((), ()))
    w = w_ref[...]
    inv = 1.0 / jnp.where(w == 0.0, 1.0, w)
    alo = agg_ref[0] * inv
    ahi = agg_ref[1] * inv
    acc = (lax.dot_general(hh_ref[...], ws_ref[...], dn,
                           preferred_element_type=jnp.float32)
           + lax.dot_general(alo, wl_ref[...], dn,
                             preferred_element_type=jnp.float32)
           + lax.dot_general(ahi, wh_ref[...], dn,
                             preferred_element_type=jnp.float32)
           + b_ref[...])
    y = jnp.where(acc > 0.0, acc, 0.01 * acc)
    ss = jnp.sum(y * y, axis=1, keepdims=True)
    nrm = jnp.sqrt(ss)
    nrm = jnp.where(nrm == 0.0, 1.0, nrm)
    res = y / nrm
    o_ref[...] = res
    if aux_ref is not None:
        aux_ref[0] = res[:, 0:128]
        aux_ref[1] = res[:, 128:256]


def _tc_update(hh, agg, w, W, b, make_aux):
    grid = (_N // _BK,)
    in_specs = [
        pl.BlockSpec((_BK, _D), lambda i: (i, 0)),
        pl.BlockSpec((2, _BK, _H), lambda i: (0, i, 0)),
        pl.BlockSpec((_BK, 1), lambda i: (i, 0)),
        pl.BlockSpec((_D, _D), lambda i: (0, 0)),
        pl.BlockSpec((_D, 128), lambda i: (0, 2)),
        pl.BlockSpec((_D, 128), lambda i: (0, 3)),
        pl.BlockSpec((1, _D), lambda i: (0, 0)),
    ]
    out_shape = [jax.ShapeDtypeStruct((_N, _D), jnp.float32)]
    out_specs = [pl.BlockSpec((_BK, _D), lambda i: (i, 0))]
    if make_aux:
        out_shape.append(jax.ShapeDtypeStruct((2, _N, _H), jnp.float32))
        out_specs.append(pl.BlockSpec((2, _BK, _H), lambda i: (0, i, 0)))
    return pl.pallas_call(
        _tc_body,
        grid=grid,
        in_specs=in_specs,
        out_specs=out_specs,
        out_shape=out_shape,
    )(hh, agg, w, W, W, W, b.reshape(1, _D))


def _prep_body(h_ref, o_ref):
    o_ref[0] = h_ref[:, 0:128]
    o_ref[1] = h_ref[:, 128:256]


def _prep(h):
    return pl.pallas_call(
        _prep_body,
        grid=(_N // _BK,),
        in_specs=[pl.BlockSpec((_BK, _D), lambda i: (i, 0))],
        out_specs=pl.BlockSpec((2, _BK, _H), lambda i: (0, i, 0)),
        out_shape=jax.ShapeDtypeStruct((2, _N, _H), jnp.float32),
    )(h)


def kernel(h, edge_index, ppr_weight, W1, b1, W2, b2):
    pk3 = edge_index[0] | (edge_index[1] << 16)
    ppr3 = ppr_weight

    wpair = _sc_wsum(pk3, ppr3)
    w = (wpair[0] + wpair[1]).reshape(-1)[:_N, None]
    tbl = _prep(h)
    agg1 = _sc_agg(tbl.reshape(2 * _N, _H), pk3, ppr3)
    hh1, tbl1 = _tc_update(h, agg1, w, W1, b1, make_aux=True)
    agg2 = _sc_agg(tbl1.reshape(2 * _N, _H), pk3, ppr3)
    (hh2,) = _tc_update(hh1, agg2, w, W2, b2, make_aux=False)
    return hh2
